# baseline (speedup 1.0000x reference)
import functools

import jax
import jax.numpy as jnp
from jax import lax
from jax.experimental import pallas as pl
from jax.experimental.pallas import tpu as pltpu
from jax.experimental.pallas import tpu_sc as plsc

NC, NS, L = 2, 16, 16
NW = NC * NS
G = 128


def _node_mlp_body(x_ref, w_ref, o_ref):
    o_ref[...] = jnp.dot(x_ref[...], w_ref[...],
                         preferred_element_type=jnp.float32)


def _node_mlp(p1, pp_w, block=2000):
    n, d = p1.shape
    return pl.pallas_call(
        _node_mlp_body,
        grid=(n // block,),
        in_specs=[pl.BlockSpec((block, d), lambda i: (i, 0)),
                  pl.BlockSpec((d, d), lambda i: (0, 0))],
        out_specs=pl.BlockSpec((block, d), lambda i: (i, 0)),
        out_shape=jax.ShapeDtypeStruct((n, d), jnp.float32),
    )(p1, pp_w)


def _wcomb_body(w4_ref, ii_ref, o_ref):
    o_ref[...] = jnp.dot(w4_ref[0], ii_ref[...],
                         preferred_element_type=jnp.float32
                         ).astype(jnp.bfloat16)


def _combined_weight(w4, ii_w):
    b, d, _ = w4.shape
    return pl.pallas_call(
        _wcomb_body,
        grid=(b,),
        in_specs=[pl.BlockSpec((1, d, d), lambda i: (i, 0, 0)),
                  pl.BlockSpec((d, d), lambda i: (0, 0))],
        out_specs=pl.BlockSpec((d, d), lambda i: (0, i)),
        out_shape=jax.ShapeDtypeStruct((d, b * d), jnp.bfloat16),
    )(w4, ii_w)


def _edge_mlp_body(nb, block, x_ref, basis_ref, w_ref, o_ref):
    d = x_ref.shape[1]
    y = jnp.dot(x_ref[...].astype(jnp.bfloat16), w_ref[...],
                preferred_element_type=jnp.float32)
    acc = y[:, 0:d] * basis_ref[:, 0:1]
    for b in range(1, nb):
        acc = acc + y[:, b * d:(b + 1) * d] * basis_ref[:, b:b + 1]
    o_ref[...] = acc


def _edge_mlp(x, basis, wcomb, block=2000):
    e, d = x.shape
    nb = basis.shape[1]
    return pl.pallas_call(
        functools.partial(_edge_mlp_body, nb, block),
        grid=(e // block,),
        in_specs=[pl.BlockSpec((block, d), lambda i: (i, 0)),
                  pl.BlockSpec((block, nb), lambda i: (i, 0)),
                  pl.BlockSpec((d, nb * d), lambda i: (0, 0))],
        out_specs=pl.BlockSpec((block, d), lambda i: (i, 0)),
        out_shape=jax.ShapeDtypeStruct((e, d), jnp.float32),
    )(x, basis, wcomb)


def _combine_body(k, p_ref, o_ref):
    acc = p_ref[0]
    for i in range(1, k):
        acc = acc + p_ref[i]
    o_ref[...] = acc


def _combine(parts, block=1000):
    k, n, d = parts.shape
    return pl.pallas_call(
        functools.partial(_combine_body, k),
        grid=(n // block,),
        in_specs=[pl.BlockSpec((k, block, d), lambda i: (0, i, 0))],
        out_specs=pl.BlockSpec((block, d), lambda i: (i, 0)),
        out_shape=jax.ShapeDtypeStruct((n, d), jnp.float32),
    )(parts)


def _gather_add(idx2, p):
    e = idx2.shape[1]
    n, d = p.shape
    ngroups = e // G
    tmain = ngroups // NW
    nrem = ngroups - tmain * NW
    T = tmain + 1
    T += (3 - (T - 4) % 3) % 3
    mesh = plsc.VectorSubcoreMesh(core_axis_name="c", subcore_axis_name="s")

    @functools.partial(
        pl.kernel,
        out_type=jax.ShapeDtypeStruct((e, d), jnp.float32),
        mesh=mesh,
        scratch_types=[
            pltpu.VMEM((3, 2, G), jnp.int32),
            pltpu.VMEM((3, G, d), jnp.float32),
            pltpu.VMEM((3, G, d), jnp.float32),
            [pltpu.SemaphoreType.DMA] * 3,
            [pltpu.SemaphoreType.DMA] * 3,
            [pltpu.SemaphoreType.DMA] * 3,
        ],
    )
    def gather_kernel(idx2_hbm, p_hbm, x_hbm, ib, ri, rj, s_ix, s_g, s_w):
        wid = lax.axis_index("s") * NC + lax.axis_index("c")

        def goff(u):
            gmain = wid + NW * jnp.minimum(u, tmain - 1)
            glast = jnp.where(wid < nrem, tmain * NW + wid, gmain)
            return jnp.where(u >= tmain, glast, gmain) * G

        def issue_idx(u, b):
            pltpu.async_copy(idx2_hbm.at[:, pl.ds(goff(u), G)],
                             ib.at[b], s_ix[b])

        def wait_idx(b):
            pltpu.make_async_copy(idx2_hbm.at[:, pl.ds(0, G)],
                                  ib.at[b], s_ix[b]).wait()

        def issue_gather(b):
            pltpu.async_copy(p_hbm.at[ib.at[b, 0]], ri.at[b], s_g[b])
            pltpu.async_copy(p_hbm.at[ib.at[b, 1]], rj.at[b], s_g[b])

        def wait_gather(b):
            pltpu.make_async_copy(p_hbm.at[ib.at[b, 0]],
                                  ri.at[b], s_g[b]).wait()
            pltpu.make_async_copy(p_hbm.at[ib.at[b, 1]],
                                  rj.at[b], s_g[b]).wait()

        def add_rows(b):
            def row(r, _):
                for q in range(d // L):
                    sl = pl.ds(q * L, L)
                    plsc.addupdate(ri.at[b, r, sl], rj[b, r, sl])
                return ()
            lax.fori_loop(0, G, row, (), unroll=2)

        def issue_write(u, b):
            pltpu.async_copy(ri.at[b], x_hbm.at[pl.ds(goff(u), G), :],
                             s_w[b])

        def wait_write(b):
            pltpu.make_async_copy(ri.at[b], x_hbm.at[pl.ds(0, G), :],
                                  s_w[b]).wait()

        def body(u, b):
            b2 = (b + 2) % 3
            wait_gather(b)
            issue_idx(u + 3, b)
            wait_write(b2)
            wait_idx(b2)
            issue_gather(b2)
            add_rows(b)
            issue_write(u, b)

        issue_idx(0, 0)
        issue_idx(1, 1)
        issue_idx(2, 2)
        wait_idx(0)
        issue_gather(0)
        wait_idx(1)
        issue_gather(1)
        wait_gather(0)
        issue_idx(3, 0)
        wait_idx(2)
        issue_gather(2)
        add_rows(0)
        issue_write(0, 0)
        wait_gather(1)
        issue_idx(4, 1)
        wait_write(0)
        wait_idx(0)
        issue_gather(0)
        add_rows(1)
        issue_write(1, 1)

        def triple(v, _):
            for i in (0, 1, 2):
                body(3 * v + 2 + i, (2 + i) % 3)
            return ()

        lax.fori_loop(0, (T - 4) // 3, triple, ())
        body(T - 2, (T - 2) % 3)
        body(T - 1, (T - 1) % 3)

        wait_gather(T % 3)
        wait_gather((T + 1) % 3)
        wait_idx((T + 2) % 3)
        wait_write((T - 1) % 3)

    return gather_kernel(idx2, p)


def _scatter_add(idx_i, i1, n):
    e, d = i1.shape
    ngroups = e // G
    tmain = ngroups // NW
    nrem = ngroups - tmain * NW
    RC = 80
    nrchunks = n // RC
    mesh = plsc.VectorSubcoreMesh(core_axis_name="c", subcore_axis_name="s")

    @functools.partial(
        pl.kernel,
        out_type=jax.ShapeDtypeStruct((NC, n, d), jnp.float32),
        mesh=mesh,
        scratch_types=[
            pltpu.VMEM((2, G), jnp.int32),
            pltpu.VMEM((2, G, d), jnp.float32),
            pltpu.VMEM_SHARED((n, d), jnp.float32),
            [pltpu.SemaphoreType.DMA] * 2,
        ],
    )
    def scatter_kernel(ii_hbm, i1_hbm, out_hbm, ib, rv, acc_sh, s_l):
        cid = lax.axis_index("c")
        sid = lax.axis_index("s")
        wid = sid * NC + cid

        def zrow(r, _):
            for q in range(d // L):
                rv[0, r, pl.ds(q * L, L)] = jnp.zeros((L,), jnp.float32)
            return ()

        lax.fori_loop(0, RC, zrow, ())
        nz = (nrchunks - sid + NS - 1) // NS

        def zchunk(t, _):
            roff = (sid + NS * t) * RC
            pltpu.sync_copy(rv.at[0, pl.ds(0, RC), :],
                            acc_sh.at[pl.ds(roff, RC), :])
            return ()

        lax.fori_loop(0, nz, zchunk, ())
        plsc.subcore_barrier()

        def goff(u):
            return (wid + NW * u) * G

        def issue_loads(u, b):
            pltpu.async_copy(ii_hbm.at[pl.ds(goff(u), G)], ib.at[b], s_l[b])
            pltpu.async_copy(i1_hbm.at[pl.ds(goff(u), G), :], rv.at[b],
                             s_l[b])

        def wait_loads(b):
            pltpu.make_async_copy(ii_hbm.at[pl.ds(0, G)],
                                  ib.at[b], s_l[b]).wait()
            pltpu.make_async_copy(i1_hbm.at[pl.ds(0, G), :],
                                  rv.at[b], s_l[b]).wait()

        def scat(b):
            pltpu.sync_copy(rv.at[b], acc_sh.at[ib.at[b]], add=True)

        issue_loads(0, 0)

        def pair(v, _):
            for b in (0, 1):
                u = 2 * v + b
                wait_loads(b)
                issue_loads(u + 1, 1 - b)
                scat(b)
            return ()

        lax.fori_loop(0, tmain // 2, pair, ())
        bl = (tmain - 1) % 2
        wait_loads(bl)
        scat(bl)

        @pl.when(wid < nrem)
        def _():
            off = (tmain * NW + wid) * G
            pltpu.sync_copy(ii_hbm.at[pl.ds(off, G)], ib.at[0])
            pltpu.async_copy(i1_hbm.at[pl.ds(off, G), :],
                             rv.at[0], s_l[0]).wait()
            pltpu.sync_copy(rv.at[0], acc_sh.at[ib.at[0]], add=True)

        plsc.subcore_barrier()

        def wchunk(t, _):
            roff = (sid + NS * t) * RC
            pltpu.sync_copy(acc_sh.at[pl.ds(roff, RC), :],
                            out_hbm.at[cid, pl.ds(roff, RC), :])
            return ()

        lax.fori_loop(0, nz, wchunk, ())

    return scatter_kernel(idx_i, i1)


def kernel(idx_i, idx_j, p1, basis, pp_w, pi_w, ii_w):
    n, d = p1.shape
    e = idx_i.shape[0]
    nb = basis.shape[1]
    idx_i = idx_i.astype(jnp.int32)
    idx_j = idx_j.astype(jnp.int32)
    idx2 = jnp.stack([idx_i, idx_j])

    p = _node_mlp(p1, pp_w)
    w4 = pi_w.reshape(d, d, nb).transpose(2, 0, 1)
    wcomb = _combined_weight(w4, ii_w)

    h = e // 2
    i1s, partss = [], []
    for c in range(2):
        sl = slice(c * h, (c + 1) * h)
        x_c = _gather_add(idx2[:, sl], p)
        i1_c = _edge_mlp(x_c, basis[sl], wcomb)
        partss.append(_scatter_add(idx_i[sl], i1_c, n))
        i1s.append(i1_c)
    i1 = jnp.concatenate(i1s)
    parts = jnp.concatenate(partss)
    p_out = _combine(parts)
    return (p_out.reshape(n, 1, d), i1.reshape(e, 1, d))


# speedup vs baseline: 1.0724x; 1.0724x over previous
import functools

import jax
import jax.numpy as jnp
from jax import lax
from jax.experimental import pallas as pl
from jax.experimental.pallas import tpu as pltpu
from jax.experimental.pallas import tpu_sc as plsc

NC, NS, L = 2, 16, 16
NW = NC * NS
G = 128


def _node_mlp_body(x_ref, w_ref, o_ref):
    o_ref[...] = jnp.dot(x_ref[...], w_ref[...],
                         preferred_element_type=jnp.float32)


def _node_mlp(p1, pp_w, block=2000):
    n, d = p1.shape
    return pl.pallas_call(
        _node_mlp_body,
        grid=(n // block,),
        in_specs=[pl.BlockSpec((block, d), lambda i: (i, 0)),
                  pl.BlockSpec((d, d), lambda i: (0, 0))],
        out_specs=pl.BlockSpec((block, d), lambda i: (i, 0)),
        out_shape=jax.ShapeDtypeStruct((n, d), jnp.float32),
    )(p1, pp_w)


def _wcomb_body(w4_ref, ii_ref, o_ref):
    o_ref[...] = jnp.dot(w4_ref[0], ii_ref[...],
                         preferred_element_type=jnp.float32
                         ).astype(jnp.bfloat16)


def _combined_weight(w4, ii_w):
    b, d, _ = w4.shape
    return pl.pallas_call(
        _wcomb_body,
        grid=(b,),
        in_specs=[pl.BlockSpec((1, d, d), lambda i: (i, 0, 0)),
                  pl.BlockSpec((d, d), lambda i: (0, 0))],
        out_specs=pl.BlockSpec((d, d), lambda i: (0, i)),
        out_shape=jax.ShapeDtypeStruct((d, b * d), jnp.bfloat16),
    )(w4, ii_w)


def _edge_mlp_body(nb, block, x_ref, basis_ref, w_ref, o_ref):
    d = x_ref.shape[1]
    y = jnp.dot(x_ref[...].astype(jnp.bfloat16), w_ref[...],
                preferred_element_type=jnp.float32)
    acc = y[:, 0:d] * basis_ref[:, 0:1]
    for b in range(1, nb):
        acc = acc + y[:, b * d:(b + 1) * d] * basis_ref[:, b:b + 1]
    o_ref[...] = acc


def _edge_mlp(x, basis, wcomb, block=2000):
    e, d = x.shape
    nb = basis.shape[1]
    return pl.pallas_call(
        functools.partial(_edge_mlp_body, nb, block),
        grid=(e // block,),
        in_specs=[pl.BlockSpec((block, d), lambda i: (i, 0)),
                  pl.BlockSpec((block, nb), lambda i: (i, 0)),
                  pl.BlockSpec((d, nb * d), lambda i: (0, 0))],
        out_specs=pl.BlockSpec((block, d), lambda i: (i, 0)),
        out_shape=jax.ShapeDtypeStruct((e, d), jnp.float32),
    )(x, basis, wcomb)


def _combine_body(k, p_ref, o_ref):
    acc = p_ref[0]
    for i in range(1, k):
        acc = acc + p_ref[i]
    o_ref[...] = acc


def _combine(parts, block=1000):
    k, n, d = parts.shape
    return pl.pallas_call(
        functools.partial(_combine_body, k),
        grid=(n // block,),
        in_specs=[pl.BlockSpec((k, block, d), lambda i: (0, i, 0))],
        out_specs=pl.BlockSpec((block, d), lambda i: (i, 0)),
        out_shape=jax.ShapeDtypeStruct((n, d), jnp.float32),
    )(parts)


def _gather_add(idx2, p):
    e = idx2.shape[1]
    n, d = p.shape
    ngroups = e // G
    tmain = ngroups // NW
    nrem = ngroups - tmain * NW
    T = tmain + 1
    T += (3 - (T - 4) % 3) % 3
    mesh = plsc.VectorSubcoreMesh(core_axis_name="c", subcore_axis_name="s")

    @functools.partial(
        pl.kernel,
        out_type=jax.ShapeDtypeStruct((e, d), jnp.float32),
        mesh=mesh,
        scratch_types=[
            pltpu.VMEM((3, 2, G), jnp.int32),
            pltpu.VMEM((3, G, d), jnp.float32),
            pltpu.VMEM((3, G, d), jnp.float32),
            [pltpu.SemaphoreType.DMA] * 3,
            [pltpu.SemaphoreType.DMA] * 3,
            [pltpu.SemaphoreType.DMA] * 3,
        ],
    )
    def gather_kernel(idx2_hbm, p_hbm, x_hbm, ib, ri, rj, s_ix, s_g, s_w):
        wid = lax.axis_index("s") * NC + lax.axis_index("c")

        def goff(u):
            gmain = wid + NW * jnp.minimum(u, tmain - 1)
            glast = jnp.where(wid < nrem, tmain * NW + wid, gmain)
            return jnp.where(u >= tmain, glast, gmain) * G

        def issue_idx(u, b):
            pltpu.async_copy(idx2_hbm.at[:, pl.ds(goff(u), G)],
                             ib.at[b], s_ix[b])

        def wait_idx(b):
            pltpu.make_async_copy(idx2_hbm.at[:, pl.ds(0, G)],
                                  ib.at[b], s_ix[b]).wait()

        def issue_gather(b):
            pltpu.async_copy(p_hbm.at[ib.at[b, 0]], ri.at[b], s_g[b])
            pltpu.async_copy(p_hbm.at[ib.at[b, 1]], rj.at[b], s_g[b])

        def wait_gather(b):
            pltpu.make_async_copy(p_hbm.at[ib.at[b, 0]],
                                  ri.at[b], s_g[b]).wait()
            pltpu.make_async_copy(p_hbm.at[ib.at[b, 1]],
                                  rj.at[b], s_g[b]).wait()

        def add_rows(b):
            def row(r, _):
                for q in range(d // L):
                    sl = pl.ds(q * L, L)
                    plsc.addupdate(ri.at[b, r, sl], rj[b, r, sl])
                return ()
            lax.fori_loop(0, G, row, (), unroll=2)

        def issue_write(u, b):
            pltpu.async_copy(ri.at[b], x_hbm.at[pl.ds(goff(u), G), :],
                             s_w[b])

        def wait_write(b):
            pltpu.make_async_copy(ri.at[b], x_hbm.at[pl.ds(0, G), :],
                                  s_w[b]).wait()

        def body(u, b):
            b2 = (b + 2) % 3
            wait_gather(b)
            issue_idx(u + 3, b)
            wait_write(b2)
            wait_idx(b2)
            issue_gather(b2)
            add_rows(b)
            issue_write(u, b)

        issue_idx(0, 0)
        issue_idx(1, 1)
        issue_idx(2, 2)
        wait_idx(0)
        issue_gather(0)
        wait_idx(1)
        issue_gather(1)
        wait_gather(0)
        issue_idx(3, 0)
        wait_idx(2)
        issue_gather(2)
        add_rows(0)
        issue_write(0, 0)
        wait_gather(1)
        issue_idx(4, 1)
        wait_write(0)
        wait_idx(0)
        issue_gather(0)
        add_rows(1)
        issue_write(1, 1)

        def triple(v, _):
            for i in (0, 1, 2):
                body(3 * v + 2 + i, (2 + i) % 3)
            return ()

        lax.fori_loop(0, (T - 4) // 3, triple, ())
        body(T - 2, (T - 2) % 3)
        body(T - 1, (T - 1) % 3)

        wait_gather(T % 3)
        wait_gather((T + 1) % 3)
        wait_idx((T + 2) % 3)
        wait_write((T - 1) % 3)

    return gather_kernel(idx2, p)


def _scatter_add(idx_i, i1, n):
    e, d = i1.shape
    ngroups = e // G
    tmain = ngroups // NW
    nrem = ngroups - tmain * NW
    RC = 80
    nrchunks = n // RC
    mesh = plsc.VectorSubcoreMesh(core_axis_name="c", subcore_axis_name="s")

    @functools.partial(
        pl.kernel,
        out_type=jax.ShapeDtypeStruct((NC, n, d), jnp.float32),
        mesh=mesh,
        scratch_types=[
            pltpu.VMEM((2, G), jnp.int32),
            pltpu.VMEM((2, G, d), jnp.float32),
            pltpu.VMEM_SHARED((n, d), jnp.float32),
            [pltpu.SemaphoreType.DMA] * 2,
        ],
    )
    def scatter_kernel(ii_hbm, i1_hbm, out_hbm, ib, rv, acc_sh, s_l):
        cid = lax.axis_index("c")
        sid = lax.axis_index("s")
        wid = sid * NC + cid

        def zrow(r, _):
            for q in range(d // L):
                rv[0, r, pl.ds(q * L, L)] = jnp.zeros((L,), jnp.float32)
            return ()

        lax.fori_loop(0, RC, zrow, ())
        nz = (nrchunks - sid + NS - 1) // NS

        def zchunk(t, _):
            roff = (sid + NS * t) * RC
            pltpu.sync_copy(rv.at[0, pl.ds(0, RC), :],
                            acc_sh.at[pl.ds(roff, RC), :])
            return ()

        lax.fori_loop(0, nz, zchunk, ())
        plsc.subcore_barrier()

        def goff(u):
            return (wid + NW * u) * G

        def issue_loads(u, b):
            pltpu.async_copy(ii_hbm.at[pl.ds(goff(u), G)], ib.at[b], s_l[b])
            pltpu.async_copy(i1_hbm.at[pl.ds(goff(u), G), :], rv.at[b],
                             s_l[b])

        def wait_loads(b):
            pltpu.make_async_copy(ii_hbm.at[pl.ds(0, G)],
                                  ib.at[b], s_l[b]).wait()
            pltpu.make_async_copy(i1_hbm.at[pl.ds(0, G), :],
                                  rv.at[b], s_l[b]).wait()

        def scat(b):
            pltpu.sync_copy(rv.at[b], acc_sh.at[ib.at[b]], add=True)

        issue_loads(0, 0)

        def pair(v, _):
            for b in (0, 1):
                u = 2 * v + b
                wait_loads(b)
                issue_loads(u + 1, 1 - b)
                scat(b)
            return ()

        lax.fori_loop(0, tmain // 2, pair, ())
        bl = (tmain - 1) % 2
        wait_loads(bl)
        scat(bl)

        @pl.when(wid < nrem)
        def _():
            off = (tmain * NW + wid) * G
            pltpu.sync_copy(ii_hbm.at[pl.ds(off, G)], ib.at[0])
            pltpu.async_copy(i1_hbm.at[pl.ds(off, G), :],
                             rv.at[0], s_l[0]).wait()
            pltpu.sync_copy(rv.at[0], acc_sh.at[ib.at[0]], add=True)

        plsc.subcore_barrier()

        def wchunk(t, _):
            roff = (sid + NS * t) * RC
            pltpu.sync_copy(acc_sh.at[pl.ds(roff, RC), :],
                            out_hbm.at[cid, pl.ds(roff, RC), :])
            return ()

        lax.fori_loop(0, nz, wchunk, ())

    return scatter_kernel(idx_i, i1)


def kernel(idx_i, idx_j, p1, basis, pp_w, pi_w, ii_w):
    n, d = p1.shape
    e = idx_i.shape[0]
    nb = basis.shape[1]
    idx_i = idx_i.astype(jnp.int32)
    idx_j = idx_j.astype(jnp.int32)
    idx2 = jnp.stack([idx_i, idx_j])

    p = _node_mlp(p1, pp_w)
    w4 = pi_w.reshape(d, d, nb).transpose(2, 0, 1)
    wcomb = _combined_weight(w4, ii_w)

    x = _gather_add(idx2, p)
    i1 = _edge_mlp(x, basis, wcomb)
    parts = _scatter_add(idx_i, i1, n)
    p_out = _combine(parts)
    return (p_out.reshape(n, 1, d), i1.reshape(e, 1, d))


# speedup vs baseline: 1.1494x; 1.0718x over previous
import functools

import jax
import jax.numpy as jnp
from jax import lax
from jax.experimental import pallas as pl
from jax.experimental.pallas import tpu as pltpu
from jax.experimental.pallas import tpu_sc as plsc

NC, NS, L = 2, 16, 16
NW = NC * NS
G = 128


def _node_mlp_body(x_ref, w_ref, o_ref):
    o_ref[...] = jnp.dot(x_ref[...], w_ref[...],
                         preferred_element_type=jnp.float32)


def _node_mlp(p1, pp_w, block=2000):
    n, d = p1.shape
    return pl.pallas_call(
        _node_mlp_body,
        grid=(n // block,),
        in_specs=[pl.BlockSpec((block, d), lambda i: (i, 0)),
                  pl.BlockSpec((d, d), lambda i: (0, 0))],
        out_specs=pl.BlockSpec((block, d), lambda i: (i, 0)),
        out_shape=jax.ShapeDtypeStruct((n, d), jnp.float32),
    )(p1, pp_w)


def _wcomb_body(w4_ref, ii_ref, o_ref):
    o_ref[...] = jnp.dot(w4_ref[0], ii_ref[...],
                         preferred_element_type=jnp.float32
                         ).astype(jnp.bfloat16)


def _combined_weight(w4, ii_w):
    b, d, _ = w4.shape
    return pl.pallas_call(
        _wcomb_body,
        grid=(b,),
        in_specs=[pl.BlockSpec((1, d, d), lambda i: (i, 0, 0)),
                  pl.BlockSpec((d, d), lambda i: (0, 0))],
        out_specs=pl.BlockSpec((d, d), lambda i: (0, i)),
        out_shape=jax.ShapeDtypeStruct((d, b * d), jnp.bfloat16),
    )(w4, ii_w)


def _edge_mlp_body(nb, block, x_ref, basis_ref, w_ref, o_ref):
    d = x_ref.shape[1]
    y = jnp.dot(x_ref[...].astype(jnp.bfloat16), w_ref[...],
                preferred_element_type=jnp.float32)
    acc = y[:, 0:d] * basis_ref[:, 0:1]
    for b in range(1, nb):
        acc = acc + y[:, b * d:(b + 1) * d] * basis_ref[:, b:b + 1]
    o_ref[...] = acc


def _edge_mlp(x, basis, wcomb, block=4000):
    e, d = x.shape
    nb = basis.shape[1]
    return pl.pallas_call(
        functools.partial(_edge_mlp_body, nb, block),
        grid=(e // block,),
        in_specs=[pl.BlockSpec((block, d), lambda i: (i, 0)),
                  pl.BlockSpec((block, nb), lambda i: (i, 0)),
                  pl.BlockSpec((d, nb * d), lambda i: (0, 0))],
        out_specs=pl.BlockSpec((block, d), lambda i: (i, 0)),
        out_shape=jax.ShapeDtypeStruct((e, d), jnp.float32),
    )(x, basis, wcomb)


def _combine_body(k, p_ref, o_ref):
    acc = p_ref[0]
    for i in range(1, k):
        acc = acc + p_ref[i]
    o_ref[...] = acc


def _combine(parts, block=1000):
    k, n, d = parts.shape
    return pl.pallas_call(
        functools.partial(_combine_body, k),
        grid=(n // block,),
        in_specs=[pl.BlockSpec((k, block, d), lambda i: (0, i, 0))],
        out_specs=pl.BlockSpec((block, d), lambda i: (i, 0)),
        out_shape=jax.ShapeDtypeStruct((n, d), jnp.float32),
    )(parts)


def _gather_add(idx2, p):
    e = idx2.shape[1]
    n, d = p.shape
    ngroups = e // G
    tmain = ngroups // NW
    nrem = ngroups - tmain * NW
    T = tmain + 1
    T += (3 - (T - 4) % 3) % 3
    mesh = plsc.VectorSubcoreMesh(core_axis_name="c", subcore_axis_name="s")

    @functools.partial(
        pl.kernel,
        out_type=jax.ShapeDtypeStruct((e, d), jnp.float32),
        mesh=mesh,
        scratch_types=[
            pltpu.VMEM((3, 2, G), jnp.int32),
            pltpu.VMEM((3, G, d), jnp.float32),
            pltpu.VMEM((3, G, d), jnp.float32),
            [pltpu.SemaphoreType.DMA] * 3,
            [pltpu.SemaphoreType.DMA] * 3,
            [pltpu.SemaphoreType.DMA] * 3,
        ],
    )
    def gather_kernel(idx2_hbm, p_hbm, x_hbm, ib, ri, rj, s_ix, s_g, s_w):
        wid = lax.axis_index("s") * NC + lax.axis_index("c")

        def goff(u):
            gmain = wid + NW * jnp.minimum(u, tmain - 1)
            glast = jnp.where(wid < nrem, tmain * NW + wid, gmain)
            return jnp.where(u >= tmain, glast, gmain) * G

        def issue_idx(u, b):
            pltpu.async_copy(idx2_hbm.at[:, pl.ds(goff(u), G)],
                             ib.at[b], s_ix[b])

        def wait_idx(b):
            pltpu.make_async_copy(idx2_hbm.at[:, pl.ds(0, G)],
                                  ib.at[b], s_ix[b]).wait()

        def issue_gather(b):
            pltpu.async_copy(p_hbm.at[ib.at[b, 0]], ri.at[b], s_g[b])
            pltpu.async_copy(p_hbm.at[ib.at[b, 1]], rj.at[b], s_g[b])

        def wait_gather(b):
            pltpu.make_async_copy(p_hbm.at[ib.at[b, 0]],
                                  ri.at[b], s_g[b]).wait()
            pltpu.make_async_copy(p_hbm.at[ib.at[b, 1]],
                                  rj.at[b], s_g[b]).wait()

        def add_rows(b):
            def row(r, _):
                for q in range(d // L):
                    sl = pl.ds(q * L, L)
                    plsc.addupdate(ri.at[b, r, sl], rj[b, r, sl])
                return ()
            lax.fori_loop(0, G, row, (), unroll=2)

        def issue_write(u, b):
            pltpu.async_copy(ri.at[b], x_hbm.at[pl.ds(goff(u), G), :],
                             s_w[b])

        def wait_write(b):
            pltpu.make_async_copy(ri.at[b], x_hbm.at[pl.ds(0, G), :],
                                  s_w[b]).wait()

        def body(u, b):
            b2 = (b + 2) % 3
            wait_gather(b)
            issue_idx(u + 3, b)
            wait_write(b2)
            wait_idx(b2)
            issue_gather(b2)
            add_rows(b)
            issue_write(u, b)

        issue_idx(0, 0)
        issue_idx(1, 1)
        issue_idx(2, 2)
        wait_idx(0)
        issue_gather(0)
        wait_idx(1)
        issue_gather(1)
        wait_gather(0)
        issue_idx(3, 0)
        wait_idx(2)
        issue_gather(2)
        add_rows(0)
        issue_write(0, 0)
        wait_gather(1)
        issue_idx(4, 1)
        wait_write(0)
        wait_idx(0)
        issue_gather(0)
        add_rows(1)
        issue_write(1, 1)

        def triple(v, _):
            for i in (0, 1, 2):
                body(3 * v + 2 + i, (2 + i) % 3)
            return ()

        lax.fori_loop(0, (T - 4) // 3, triple, ())
        body(T - 2, (T - 2) % 3)
        body(T - 1, (T - 1) % 3)

        wait_gather(T % 3)
        wait_gather((T + 1) % 3)
        wait_idx((T + 2) % 3)
        wait_write((T - 1) % 3)

    return gather_kernel(idx2, p)


def _scatter_add(idx_i, i1, n):
    e, d = i1.shape
    ngroups = e // G
    tmain = ngroups // NW
    nrem = ngroups - tmain * NW
    RC = 80
    nrchunks = n // RC
    mesh = plsc.VectorSubcoreMesh(core_axis_name="c", subcore_axis_name="s")

    @functools.partial(
        pl.kernel,
        out_type=jax.ShapeDtypeStruct((NC, n, d), jnp.float32),
        mesh=mesh,
        scratch_types=[
            pltpu.VMEM((2, G), jnp.int32),
            pltpu.VMEM((2, G, d), jnp.float32),
            pltpu.VMEM_SHARED((n, d), jnp.float32),
            [pltpu.SemaphoreType.DMA] * 2,
        ],
    )
    def scatter_kernel(ii_hbm, i1_hbm, out_hbm, ib, rv, acc_sh, s_l):
        cid = lax.axis_index("c")
        sid = lax.axis_index("s")
        wid = sid * NC + cid

        def zrow(r, _):
            for q in range(d // L):
                rv[0, r, pl.ds(q * L, L)] = jnp.zeros((L,), jnp.float32)
            return ()

        lax.fori_loop(0, RC, zrow, ())
        nz = (nrchunks - sid + NS - 1) // NS

        def zchunk(t, _):
            roff = (sid + NS * t) * RC
            pltpu.sync_copy(rv.at[0, pl.ds(0, RC), :],
                            acc_sh.at[pl.ds(roff, RC), :])
            return ()

        lax.fori_loop(0, nz, zchunk, ())
        plsc.subcore_barrier()

        def goff(u):
            return (wid + NW * u) * G

        def issue_loads(u, b):
            pltpu.async_copy(ii_hbm.at[pl.ds(goff(u), G)], ib.at[b], s_l[b])
            pltpu.async_copy(i1_hbm.at[pl.ds(goff(u), G), :], rv.at[b],
                             s_l[b])

        def wait_loads(b):
            pltpu.make_async_copy(ii_hbm.at[pl.ds(0, G)],
                                  ib.at[b], s_l[b]).wait()
            pltpu.make_async_copy(i1_hbm.at[pl.ds(0, G), :],
                                  rv.at[b], s_l[b]).wait()

        def scat(b):
            pltpu.sync_copy(rv.at[b], acc_sh.at[ib.at[b]], add=True)

        issue_loads(0, 0)

        def pair(v, _):
            for b in (0, 1):
                u = 2 * v + b
                wait_loads(b)
                issue_loads(u + 1, 1 - b)
                scat(b)
            return ()

        lax.fori_loop(0, tmain // 2, pair, ())
        bl = (tmain - 1) % 2
        wait_loads(bl)
        scat(bl)

        @pl.when(wid < nrem)
        def _():
            off = (tmain * NW + wid) * G
            pltpu.sync_copy(ii_hbm.at[pl.ds(off, G)], ib.at[0])
            pltpu.async_copy(i1_hbm.at[pl.ds(off, G), :],
                             rv.at[0], s_l[0]).wait()
            pltpu.sync_copy(rv.at[0], acc_sh.at[ib.at[0]], add=True)

        plsc.subcore_barrier()

        def wchunk(t, _):
            roff = (sid + NS * t) * RC
            pltpu.sync_copy(acc_sh.at[pl.ds(roff, RC), :],
                            out_hbm.at[cid, pl.ds(roff, RC), :])
            return ()

        lax.fori_loop(0, nz, wchunk, ())

    return scatter_kernel(idx_i, i1)


def kernel(idx_i, idx_j, p1, basis, pp_w, pi_w, ii_w):
    n, d = p1.shape
    e = idx_i.shape[0]
    nb = basis.shape[1]
    idx_i = idx_i.astype(jnp.int32)
    idx_j = idx_j.astype(jnp.int32)
    idx2 = jnp.stack([idx_i, idx_j])

    p = _node_mlp(p1, pp_w)
    w4 = pi_w.reshape(d, d, nb).transpose(2, 0, 1)
    wcomb = _combined_weight(w4, ii_w)

    x = _gather_add(idx2, p)
    i1 = _edge_mlp(x, basis, wcomb)
    parts = _scatter_add(idx_i, i1, n)
    p_out = _combine(parts)
    return (p_out.reshape(n, 1, d), i1.reshape(e, 1, d))


# speedup vs baseline: 1.1871x; 1.0329x over previous
import functools

import jax
import jax.numpy as jnp
from jax import lax
from jax.experimental import pallas as pl
from jax.experimental.pallas import tpu as pltpu
from jax.experimental.pallas import tpu_sc as plsc

NC, NS, L = 2, 16, 16
NW = NC * NS
G = 128


def _node_mlp_body(x_ref, w_ref, o_ref):
    o_ref[...] = jnp.dot(x_ref[...], w_ref[...],
                         preferred_element_type=jnp.float32)


def _node_mlp(p1, pp_w, block=2000):
    n, d = p1.shape
    return pl.pallas_call(
        _node_mlp_body,
        grid=(n // block,),
        in_specs=[pl.BlockSpec((block, d), lambda i: (i, 0)),
                  pl.BlockSpec((d, d), lambda i: (0, 0))],
        out_specs=pl.BlockSpec((block, d), lambda i: (i, 0)),
        out_shape=jax.ShapeDtypeStruct((n, d), jnp.float32),
    )(p1, pp_w)


def _wcomb_body(w4_ref, ii_ref, o_ref):
    o_ref[...] = jnp.dot(w4_ref[0], ii_ref[...],
                         preferred_element_type=jnp.float32
                         ).astype(jnp.bfloat16)


def _combined_weight(w4, ii_w):
    b, d, _ = w4.shape
    return pl.pallas_call(
        _wcomb_body,
        grid=(b,),
        in_specs=[pl.BlockSpec((1, d, d), lambda i: (i, 0, 0)),
                  pl.BlockSpec((d, d), lambda i: (0, 0))],
        out_specs=pl.BlockSpec((d, d), lambda i: (0, i)),
        out_shape=jax.ShapeDtypeStruct((d, b * d), jnp.bfloat16),
    )(w4, ii_w)


def _edge_mlp_body(nb, block, x_ref, basis_ref, w_ref, o_ref):
    d = x_ref.shape[1]
    y = jnp.dot(x_ref[...].astype(jnp.bfloat16), w_ref[...],
                preferred_element_type=jnp.float32)
    acc = y[:, 0:d] * basis_ref[:, 0:1]
    for b in range(1, nb):
        acc = acc + y[:, b * d:(b + 1) * d] * basis_ref[:, b:b + 1]
    o_ref[...] = acc


def _edge_mlp(x, basis, wcomb, block=8000):
    e, d = x.shape
    nb = basis.shape[1]
    return pl.pallas_call(
        functools.partial(_edge_mlp_body, nb, block),
        grid=(e // block,),
        in_specs=[pl.BlockSpec((block, d), lambda i: (i, 0)),
                  pl.BlockSpec((block, nb), lambda i: (i, 0)),
                  pl.BlockSpec((d, nb * d), lambda i: (0, 0))],
        out_specs=pl.BlockSpec((block, d), lambda i: (i, 0)),
        out_shape=jax.ShapeDtypeStruct((e, d), jnp.float32),
    )(x, basis, wcomb)


def _combine_body(k, p_ref, o_ref):
    acc = p_ref[0]
    for i in range(1, k):
        acc = acc + p_ref[i]
    o_ref[...] = acc


def _combine(parts, block=2000):
    k, n, d = parts.shape
    return pl.pallas_call(
        functools.partial(_combine_body, k),
        grid=(n // block,),
        in_specs=[pl.BlockSpec((k, block, d), lambda i: (0, i, 0))],
        out_specs=pl.BlockSpec((block, d), lambda i: (i, 0)),
        out_shape=jax.ShapeDtypeStruct((n, d), jnp.float32),
    )(parts)


def _gather_add(idx2, p):
    e = idx2.shape[1]
    n, d = p.shape
    ngroups = e // G
    tmain = ngroups // NW
    nrem = ngroups - tmain * NW
    T = tmain + 1
    T += (3 - (T - 4) % 3) % 3
    mesh = plsc.VectorSubcoreMesh(core_axis_name="c", subcore_axis_name="s")

    @functools.partial(
        pl.kernel,
        out_type=jax.ShapeDtypeStruct((e, d), jnp.float32),
        mesh=mesh,
        scratch_types=[
            pltpu.VMEM((3, 2, G), jnp.int32),
            pltpu.VMEM((3, G, d), jnp.float32),
            pltpu.VMEM((3, G, d), jnp.float32),
            [pltpu.SemaphoreType.DMA] * 3,
            [pltpu.SemaphoreType.DMA] * 3,
            [pltpu.SemaphoreType.DMA] * 3,
        ],
    )
    def gather_kernel(idx2_hbm, p_hbm, x_hbm, ib, ri, rj, s_ix, s_g, s_w):
        wid = lax.axis_index("s") * NC + lax.axis_index("c")

        def goff(u):
            gmain = wid + NW * jnp.minimum(u, tmain - 1)
            glast = jnp.where(wid < nrem, tmain * NW + wid, gmain)
            return jnp.where(u >= tmain, glast, gmain) * G

        def issue_idx(u, b):
            pltpu.async_copy(idx2_hbm.at[:, pl.ds(goff(u), G)],
                             ib.at[b], s_ix[b])

        def wait_idx(b):
            pltpu.make_async_copy(idx2_hbm.at[:, pl.ds(0, G)],
                                  ib.at[b], s_ix[b]).wait()

        def issue_gather(b):
            pltpu.async_copy(p_hbm.at[ib.at[b, 0]], ri.at[b], s_g[b])
            pltpu.async_copy(p_hbm.at[ib.at[b, 1]], rj.at[b], s_g[b])

        def wait_gather(b):
            pltpu.make_async_copy(p_hbm.at[ib.at[b, 0]],
                                  ri.at[b], s_g[b]).wait()
            pltpu.make_async_copy(p_hbm.at[ib.at[b, 1]],
                                  rj.at[b], s_g[b]).wait()

        def add_rows(b):
            def row(r, _):
                for q in range(d // L):
                    sl = pl.ds(q * L, L)
                    plsc.addupdate(ri.at[b, r, sl], rj[b, r, sl])
                return ()
            lax.fori_loop(0, G, row, (), unroll=2)

        def issue_write(u, b):
            pltpu.async_copy(ri.at[b], x_hbm.at[pl.ds(goff(u), G), :],
                             s_w[b])

        def wait_write(b):
            pltpu.make_async_copy(ri.at[b], x_hbm.at[pl.ds(0, G), :],
                                  s_w[b]).wait()

        def body(u, b):
            b2 = (b + 2) % 3
            wait_gather(b)
            issue_idx(u + 3, b)
            wait_write(b2)
            wait_idx(b2)
            issue_gather(b2)
            add_rows(b)
            issue_write(u, b)

        issue_idx(0, 0)
        issue_idx(1, 1)
        issue_idx(2, 2)
        wait_idx(0)
        issue_gather(0)
        wait_idx(1)
        issue_gather(1)
        wait_gather(0)
        issue_idx(3, 0)
        wait_idx(2)
        issue_gather(2)
        add_rows(0)
        issue_write(0, 0)
        wait_gather(1)
        issue_idx(4, 1)
        wait_write(0)
        wait_idx(0)
        issue_gather(0)
        add_rows(1)
        issue_write(1, 1)

        def triple(v, _):
            for i in (0, 1, 2):
                body(3 * v + 2 + i, (2 + i) % 3)
            return ()

        lax.fori_loop(0, (T - 4) // 3, triple, ())
        body(T - 2, (T - 2) % 3)
        body(T - 1, (T - 1) % 3)

        wait_gather(T % 3)
        wait_gather((T + 1) % 3)
        wait_idx((T + 2) % 3)
        wait_write((T - 1) % 3)

    return gather_kernel(idx2, p)


def _scatter_add(idx_i, i1, n):
    e, d = i1.shape
    ngroups = e // G
    tmain = ngroups // NW
    nrem = ngroups - tmain * NW
    RC = 80
    nrchunks = n // RC
    mesh = plsc.VectorSubcoreMesh(core_axis_name="c", subcore_axis_name="s")

    @functools.partial(
        pl.kernel,
        out_type=jax.ShapeDtypeStruct((NC, n, d), jnp.float32),
        mesh=mesh,
        scratch_types=[
            pltpu.VMEM((2, G), jnp.int32),
            pltpu.VMEM((2, G, d), jnp.float32),
            pltpu.VMEM_SHARED((n, d), jnp.float32),
            [pltpu.SemaphoreType.DMA] * 2,
        ],
    )
    def scatter_kernel(ii_hbm, i1_hbm, out_hbm, ib, rv, acc_sh, s_l):
        cid = lax.axis_index("c")
        sid = lax.axis_index("s")
        wid = sid * NC + cid

        def zrow(r, _):
            for q in range(d // L):
                rv[0, r, pl.ds(q * L, L)] = jnp.zeros((L,), jnp.float32)
            return ()

        lax.fori_loop(0, RC, zrow, ())
        nz = (nrchunks - sid + NS - 1) // NS

        def zchunk(t, _):
            roff = (sid + NS * t) * RC
            pltpu.sync_copy(rv.at[0, pl.ds(0, RC), :],
                            acc_sh.at[pl.ds(roff, RC), :])
            return ()

        lax.fori_loop(0, nz, zchunk, ())
        plsc.subcore_barrier()

        def goff(u):
            return (wid + NW * u) * G

        def issue_loads(u, b):
            pltpu.async_copy(ii_hbm.at[pl.ds(goff(u), G)], ib.at[b], s_l[b])
            pltpu.async_copy(i1_hbm.at[pl.ds(goff(u), G), :], rv.at[b],
                             s_l[b])

        def wait_loads(b):
            pltpu.make_async_copy(ii_hbm.at[pl.ds(0, G)],
                                  ib.at[b], s_l[b]).wait()
            pltpu.make_async_copy(i1_hbm.at[pl.ds(0, G), :],
                                  rv.at[b], s_l[b]).wait()

        def scat(b):
            pltpu.sync_copy(rv.at[b], acc_sh.at[ib.at[b]], add=True)

        issue_loads(0, 0)

        def pair(v, _):
            for b in (0, 1):
                u = 2 * v + b
                wait_loads(b)
                issue_loads(u + 1, 1 - b)
                scat(b)
            return ()

        lax.fori_loop(0, tmain // 2, pair, ())
        bl = (tmain - 1) % 2
        wait_loads(bl)
        scat(bl)

        @pl.when(wid < nrem)
        def _():
            off = (tmain * NW + wid) * G
            pltpu.sync_copy(ii_hbm.at[pl.ds(off, G)], ib.at[0])
            pltpu.async_copy(i1_hbm.at[pl.ds(off, G), :],
                             rv.at[0], s_l[0]).wait()
            pltpu.sync_copy(rv.at[0], acc_sh.at[ib.at[0]], add=True)

        plsc.subcore_barrier()

        def wchunk(t, _):
            roff = (sid + NS * t) * RC
            pltpu.sync_copy(acc_sh.at[pl.ds(roff, RC), :],
                            out_hbm.at[cid, pl.ds(roff, RC), :])
            return ()

        lax.fori_loop(0, nz, wchunk, ())

    return scatter_kernel(idx_i, i1)


def kernel(idx_i, idx_j, p1, basis, pp_w, pi_w, ii_w):
    n, d = p1.shape
    e = idx_i.shape[0]
    nb = basis.shape[1]
    idx_i = idx_i.astype(jnp.int32)
    idx_j = idx_j.astype(jnp.int32)
    idx2 = jnp.stack([idx_i, idx_j])

    p = _node_mlp(p1, pp_w)
    w4 = pi_w.reshape(d, d, nb).transpose(2, 0, 1)
    wcomb = _combined_weight(w4, ii_w)

    x = _gather_add(idx2, p)
    i1 = _edge_mlp(x, basis, wcomb)
    parts = _scatter_add(idx_i, i1, n)
    p_out = _combine(parts)
    return (p_out.reshape(n, 1, d), i1.reshape(e, 1, d))
